# baseline (speedup 1.0000x reference)
import functools

import jax
import jax.numpy as jnp
from jax import lax
from jax.experimental import pallas as pl
from jax.experimental.pallas import tpu as pltpu
from jax.experimental.pallas import tpu_sc as plsc

NUM_EMBEDDINGS = 8192
EMBEDDING_DIM = 768
POS_OFFSET = 2
BATCH = 4
SEQ_LEN = 8192

_V = NUM_EMBEDDINGS + POS_OFFSET
_NW = 32
_SPAN = SEQ_LEN // _NW
_G = 64
_NCH = _SPAN // _G
_NBUF = 2


def _pid_body(mask_ref, idx_ref, flag_ref):
    m = mask_ref[...]
    s = m
    sh = 1
    while sh < SEQ_LEN:
        zeros = jnp.zeros((BATCH, sh), jnp.int32)
        s = s + jnp.concatenate([zeros, s[:, : SEQ_LEN - sh]], axis=1)
        sh *= 2
    ids = s * m + 1
    ids = jnp.minimum(jnp.maximum(ids, 0), _V - 1)
    idx_ref[...] = ids
    shared = jnp.min((ids == ids[0:1, :]).astype(jnp.int32))
    ramp_ids = ids[0:1, 0:1] + lax.broadcasted_iota(jnp.int32, ids.shape, 1)
    ramp = jnp.min((ids == ramp_ids).astype(jnp.int32))
    li = lax.broadcasted_iota(jnp.int32, (8, 128), 1)
    si = lax.broadcasted_iota(jnp.int32, (8, 128), 0)
    lane = ((li == 0).astype(jnp.int32) * shared
            + (li == 1).astype(jnp.int32) * ramp
            + (li == 2).astype(jnp.int32) * ids[0, 0])
    flag_ref[...] = (si == 0).astype(jnp.int32) * lane


_pid = pl.pallas_call(
    _pid_body,
    out_shape=(
        jax.ShapeDtypeStruct((BATCH, SEQ_LEN), jnp.int32),
        jax.ShapeDtypeStruct((8, 128), jnp.int32),
    ),
)


def _sc_body(flag_hbm, idx_hbm, weight_hbm, out_hbm, flag_v, idx_v, rows,
             sem_r, sem_w, sem_s):
    cid = lax.axis_index("c")
    sid = lax.axis_index("s")
    wid = cid * 16 + sid

    pltpu.async_copy(idx_hbm.at[0, wid], idx_v.at[0], sem_s).wait()

    def _out_slice(b, j):
        return out_hbm.at[pl.ds(b * SEQ_LEN + wid * _SPAN + j * _G, _G)]

    def _fire_read(j):
        k = j % _NBUF
        pltpu.async_copy(weight_hbm.at[idx_v.at[0, j]],
                         rows[k].at[pl.ds(0, _G)], sem_r[k])

    def _wait_read(j):
        k = j % _NBUF
        pltpu.make_async_copy(weight_hbm.at[idx_v.at[0, j]],
                              rows[k].at[pl.ds(0, _G)], sem_r[k]).wait()

    def _fire_writes(j, roff=0):
        k = j % _NBUF
        for b in range(BATCH):
            pltpu.async_copy(rows[k].at[pl.ds(roff, _G)], _out_slice(b, j),
                             sem_w[k])

    def _wait_writes(j):
        k = j % _NBUF
        for b in range(BATCH):
            pltpu.make_async_copy(rows[k].at[pl.ds(0, _G)], _out_slice(b, j),
                                  sem_w[k]).wait()

    _fire_read(0)
    pltpu.async_copy(flag_hbm.at[0], flag_v, sem_s)
    for b in range(1, BATCH):
        pltpu.async_copy(idx_hbm.at[b, wid], idx_v.at[b], sem_s)
    pltpu.make_async_copy(flag_hbm.at[0], flag_v, sem_s).wait()
    for b in range(1, BATCH):
        pltpu.make_async_copy(idx_hbm.at[b, wid], idx_v.at[b], sem_s).wait()
    flag16 = flag_v[pl.ds(0, 16)]
    shared = flag16[0] != 0
    ramp = flag16[1] != 0
    first = flag16[2]

    def _lin_base(j):
        base = first + wid * _SPAN + j * _G
        aligned = pl.multiple_of((base // 8) * 8, 8)
        return aligned, base - aligned

    def _fire_read_lin(j):
        k = j % _NBUF
        aligned, _ = _lin_base(j)
        pltpu.async_copy(weight_hbm.at[pl.ds(aligned, _G + 8)], rows[k],
                         sem_r[k])

    def _wait_read_lin(j):
        k = j % _NBUF
        pltpu.make_async_copy(weight_hbm.at[pl.ds(0, _G + 8)], rows[k],
                              sem_r[k]).wait()

    def _ring(linear):
        lin = [False] + [linear] * (_NCH - 2) + [False]
        for j in range(_NCH):
            if lin[j]:
                _wait_read_lin(j)
                _fire_writes(j, _lin_base(j)[1])
            else:
                _wait_read(j)
                _fire_writes(j, 0)
            if j + 1 < _NCH:
                if j - 1 >= 0:
                    _wait_writes(j - 1)
                if lin[j + 1]:
                    _fire_read_lin(j + 1)
                else:
                    _fire_read(j + 1)
        for j in range(_NCH - 2, _NCH):
            _wait_writes(j)

    @pl.when(shared & ramp)
    def _fan_out_linear():
        _ring(True)

    @pl.when(shared & jnp.logical_not(ramp))
    def _fan_out():
        _ring(False)

    @pl.when(jnp.logical_not(shared))
    def _full_gather():
        row0 = rows[0].at[pl.ds(0, _G)]
        row1 = rows[1].at[pl.ds(0, _G)]
        for b in range(BATCH):
            if b > 0:
                pltpu.async_copy(weight_hbm.at[idx_v.at[b, 0]], row0,
                                 sem_r[0])

            def _step(g, carry, b=b):
                j0 = g * 2
                pltpu.async_copy(weight_hbm.at[idx_v.at[b, j0 + 1]], row1,
                                 sem_r[1])
                pltpu.make_async_copy(weight_hbm.at[idx_v.at[b, 0]], row0,
                                      sem_r[0]).wait()
                pltpu.sync_copy(row0, out_hbm.at[
                    pl.ds(b * SEQ_LEN + wid * _SPAN + j0 * _G, _G)])
                jn = jnp.minimum(j0 + 2, _NCH - 1)
                pltpu.async_copy(weight_hbm.at[idx_v.at[b, jn]], row0,
                                 sem_r[0])
                pltpu.make_async_copy(weight_hbm.at[idx_v.at[b, 0]], row1,
                                      sem_r[1]).wait()
                pltpu.sync_copy(row1, out_hbm.at[
                    pl.ds(b * SEQ_LEN + wid * _SPAN + (j0 + 1) * _G, _G)])
                return carry

            lax.fori_loop(0, _NCH // 2, _step, 0, unroll=False)
            pltpu.make_async_copy(weight_hbm.at[idx_v.at[b, 0]], row0,
                                  sem_r[0]).wait()


@functools.partial(
    pl.kernel,
    mesh=plsc.VectorSubcoreMesh(core_axis_name="c", subcore_axis_name="s"),
    compiler_params=pltpu.CompilerParams(use_tc_tiling_on_sc=False),
    out_type=jax.ShapeDtypeStruct((BATCH * SEQ_LEN, EMBEDDING_DIM), jnp.float32),
    scratch_types=[
        pltpu.VMEM((128,), jnp.int32),
        pltpu.VMEM((BATCH, _NCH, _G), jnp.int32),
        [pltpu.VMEM((_G + 8, EMBEDDING_DIM), jnp.float32)] * _NBUF,
        [pltpu.SemaphoreType.DMA] * _NBUF,
        [pltpu.SemaphoreType.DMA] * _NBUF,
        pltpu.SemaphoreType.DMA,
    ],
)
def _embed_gather(flag_hbm, idx_hbm, weight_hbm, out_hbm, flag_v, idx_v, rows,
                  sem_r, sem_w, sem_s):
    _sc_body(flag_hbm, idx_hbm, weight_hbm, out_hbm, flag_v, idx_v, rows,
             sem_r, sem_w, sem_s)


def kernel(attention_mask, past_key_values_length, weight):
    del past_key_values_length
    idx, flags = _pid(attention_mask.astype(jnp.int32))
    out = _embed_gather(flags, idx.reshape(BATCH, _NW, _NCH, _G), weight)
    return out.reshape(BATCH, SEQ_LEN, EMBEDDING_DIM)
